# baseline (speedup 1.0000x reference)
import numpy as np
import jax
import jax.numpy as jnp
from jax import lax
from jax.experimental import pallas as pl
from jax.experimental.pallas import tpu as pltpu

_FEAT_STRIDE = 16
_SCALES = np.array([8.0, 16.0, 32.0])
_RATIOS = np.array([0.5, 1.0, 2.0])
_PRE_NMS_TOPN = 6000
_POST_NMS_TOPN = 300
_NMS_THRESH = 0.5
_MIN_SIZE = 16.0
_IM_H = 512.0
_IM_W = 512.0

_H = 32
_W = 32
_A = 9
_N = _H * _W * _A
_R, _C = 72, 128
_ONE_BITS = 0x3F800000


def _np_whctrs(anchor):
    w = anchor[2] - anchor[0] + 1.0
    h = anchor[3] - anchor[1] + 1.0
    return w, h, anchor[0] + 0.5 * (w - 1.0), anchor[1] + 0.5 * (h - 1.0)


def _np_mkanchors(ws, hs, x_ctr, y_ctr):
    ws = ws[:, None]
    hs = hs[:, None]
    return np.hstack([x_ctr - 0.5 * (ws - 1.0), y_ctr - 0.5 * (hs - 1.0),
                      x_ctr + 0.5 * (ws - 1.0), y_ctr + 0.5 * (hs - 1.0)])


def _np_anchor_planes():
    base = np.array([0.0, 0.0, _FEAT_STRIDE - 1.0, _FEAT_STRIDE - 1.0])
    w, h, xc, yc = _np_whctrs(base)
    size_ratios = (w * h) / _RATIOS
    ws = np.round(np.sqrt(size_ratios))
    hs = np.round(ws * _RATIOS)
    ratio_anchors = _np_mkanchors(ws, hs, xc, yc)
    rows = []
    for i in range(ratio_anchors.shape[0]):
        w, h, xc, yc = _np_whctrs(ratio_anchors[i])
        rows.append(_np_mkanchors(w * _SCALES, h * _SCALES, xc, yc))
    base_anchors = np.vstack(rows)
    shift = np.arange(_W) * _FEAT_STRIDE
    sx, sy = np.meshgrid(shift, shift)
    shifts = np.stack([sx.ravel(), sy.ravel(), sx.ravel(), sy.ravel()], axis=1)
    anchors = (shifts[:, None, :].astype(np.float32)
               + base_anchors[None, :, :].astype(np.float32)).reshape(-1, 4)
    aw = anchors[:, 2] - anchors[:, 0] + np.float32(1.0)
    ah = anchors[:, 3] - anchors[:, 1] + np.float32(1.0)
    acx = anchors[:, 0] + np.float32(0.5) * aw
    acy = anchors[:, 1] + np.float32(0.5) * ah
    shp = (_R, _C)
    return (aw.reshape(shp), ah.reshape(shp), acx.reshape(shp), acy.reshape(shp))


_AW, _AH, _ACX, _ACY = _np_anchor_planes()
_T72 = np.tril(np.ones((_R, _R), np.float32), -1)
_TRIU = np.triu(np.ones((_C, _C), np.float32), 1)


def _nms_body(s_ref, dx_ref, dy_ref, dw_ref, dh_ref,
              aw_ref, ah_ref, acx_ref, acy_ref, t72_ref, triu_ref,
              ox1_ref, oy1_ref, ox2_ref, oy2_ref):
    aw = aw_ref[...]
    ah = ah_ref[...]
    pcx = dx_ref[...] * aw + acx_ref[...]
    pcy = dy_ref[...] * ah + acy_ref[...]
    pw = jnp.exp(dw_ref[...]) * aw
    ph = jnp.exp(dh_ref[...]) * ah
    x1 = jnp.clip(pcx - 0.5 * pw, 0.0, _IM_W - 1.0)
    y1 = jnp.clip(pcy - 0.5 * ph, 0.0, _IM_H - 1.0)
    x2 = jnp.clip(pcx + 0.5 * pw, 0.0, _IM_W - 1.0)
    y2 = jnp.clip(pcy + 0.5 * ph, 0.0, _IM_H - 1.0)
    ws = x2 - x1 + 1.0
    hs = y2 - y1 + 1.0
    valid = (ws >= _MIN_SIZE) & (hs >= _MIN_SIZE)
    s = jnp.where(valid, s_ref[...], -jnp.inf)
    area = ws * hs

    nfin = jnp.sum((s >= 0.0).astype(jnp.float32))

    def bs_body(_, carry):
        lo, hi = carry
        mid = (lo + hi) // 2
        t = lax.bitcast_convert_type(mid, jnp.float32)
        ge = jnp.sum((s >= t).astype(jnp.float32)) >= _PRE_NMS_TOPN
        return (jnp.where(ge, mid, lo), jnp.where(ge, hi, mid))

    lo, _ = lax.fori_loop(0, 31, bs_body,
                          (jnp.int32(0), jnp.int32(_ONE_BITS)))
    v = jnp.where(nfin >= _PRE_NMS_TOPN,
                  lax.bitcast_convert_type(lo, jnp.float32),
                  -jnp.inf)

    cgt = jnp.sum((s > v).astype(jnp.float32))
    eq = (s == v).astype(jnp.float32)
    rowpre = jnp.sum(
        jnp.dot(t72_ref[...], eq, preferred_element_type=jnp.float32),
        axis=1, keepdims=True)
    lanepre = jnp.dot(eq, triu_ref[...], preferred_element_type=jnp.float32)
    rank = lanepre + rowpre
    in_top = (s > v) | ((s == v) & (rank < (_PRE_NMS_TOPN - cgt)))

    dead = jnp.float32(-3e38)
    key = jnp.where(in_top, jnp.maximum(s, jnp.float32(-1e30)), dead)

    def _fold(x3, op):
        a = op(op(x3[0], x3[1]), op(x3[2], x3[3]))
        b = op(op(x3[4], x3[5]), op(x3[6], x3[7]))
        return op(op(a, b), x3[8])

    def _allred(x, op):
        for sh in (64, 32, 16, 8, 4, 2, 1):
            x = op(x, pltpu.roll(x, sh, 1))
        for sh in (4, 2, 1):
            x = op(x, pltpu.roll(x, sh, 0))
        return x

    key3 = key.reshape(9, 8, 128)
    x13 = x1.reshape(9, 8, 128)
    y13 = y1.reshape(9, 8, 128)
    x23 = x2.reshape(9, 8, 128)
    y23 = y2.reshape(9, 8, 128)
    area3 = area.reshape(9, 8, 128)
    ii3 = (lax.broadcasted_iota(jnp.int32, (9, 8, 128), 0) * 1024
           + lax.broadcasted_iota(jnp.int32, (9, 8, 128), 1) * 128
           + lax.broadcasted_iota(jnp.int32, (9, 8, 128), 2))
    oi = (lax.broadcasted_iota(jnp.int32, (8, 128), 0) * 128
          + lax.broadcasted_iota(jnp.int32, (8, 128), 1))
    zero8 = jnp.zeros((8, 128), jnp.float32)
    ninf = jnp.float32(-jnp.inf)
    inv_norm = jnp.float32(1.0) / jnp.float32(_IM_W - 1.0)
    fmax, fmin = jnp.maximum, jnp.minimum

    def sel_body(i, carry):
        key3, ox1, oy1, ox2, oy2 = carry
        mv = _allred(_fold(key3, fmax), fmax)
        foundv = mv > jnp.float32(-2e38)
        cand = jnp.where(key3 == mv[None], ii3, jnp.int32(_N))
        sv = _allred(_fold(cand, fmin), fmin)
        selm = ii3 == sv[None]
        bx1 = _allred(_fold(jnp.where(selm, x13, ninf), fmax), fmax)
        by1 = _allred(_fold(jnp.where(selm, y13, ninf), fmax), fmax)
        bx2 = _allred(_fold(jnp.where(selm, x23, ninf), fmax), fmax)
        by2 = _allred(_fold(jnp.where(selm, y23, ninf), fmax), fmax)
        barea = (bx2 - bx1 + 1.0) * (by2 - by1 + 1.0)
        iw = fmax(0.0, fmin(bx2[None], x23) - fmax(bx1[None], x13) + 1.0)
        ih = fmax(0.0, fmin(by2[None], y23) - fmax(by1[None], y13) + 1.0)
        inter = iw * ih
        iou = inter / (barea[None] + area3 - inter)
        key3 = jnp.where(iou > _NMS_THRESH, dead, key3)
        om = (oi == i) & foundv
        ox1 = jnp.where(om, bx1 * inv_norm, ox1)
        oy1 = jnp.where(om, by1 * inv_norm, oy1)
        ox2 = jnp.where(om, bx2 * inv_norm, ox2)
        oy2 = jnp.where(om, by2 * inv_norm, oy2)
        return key3, ox1, oy1, ox2, oy2

    _, ox1, oy1, ox2, oy2 = lax.fori_loop(
        0, _POST_NMS_TOPN, sel_body, (key3, zero8, zero8, zero8, zero8))
    ox1_ref[...] = ox1
    oy1_ref[...] = oy1
    ox2_ref[...] = ox2
    oy2_ref[...] = oy2


def kernel(rpn_cls_prob, rpn_bbox_pred):
    shp = (_R, _C)
    s = rpn_cls_prob[0, :, :, _A:].reshape(shp)
    deltas = rpn_bbox_pred[0].reshape(-1, 4)
    dx = deltas[:, 0].reshape(shp)
    dy = deltas[:, 1].reshape(shp)
    dw = deltas[:, 2].reshape(shp)
    dh = deltas[:, 3].reshape(shp)
    f32 = jnp.float32
    outs = pl.pallas_call(
        _nms_body,
        out_shape=[jax.ShapeDtypeStruct((8, 128), f32)] * 4,
    )(s, dx, dy, dw, dh,
      jnp.asarray(_AW), jnp.asarray(_AH), jnp.asarray(_ACX), jnp.asarray(_ACY),
      jnp.asarray(_T72), jnp.asarray(_TRIU))
    coords = [o.reshape(-1)[:_POST_NMS_TOPN] for o in outs]
    return jnp.stack(coords, axis=1)[None, :, :]


# speedup vs baseline: 2.8816x; 2.8816x over previous
import numpy as np
import jax
import jax.numpy as jnp
from jax import lax
from jax.experimental import pallas as pl
from jax.experimental.pallas import tpu as pltpu

_FEAT_STRIDE = 16
_SCALES = np.array([8.0, 16.0, 32.0])
_RATIOS = np.array([0.5, 1.0, 2.0])
_PRE_NMS_TOPN = 6000
_POST_NMS_TOPN = 300
_NMS_THRESH = 0.5
_MIN_SIZE = 16.0
_IM_H = 512.0
_IM_W = 512.0

_H = 32
_W = 32
_A = 9
_N = _H * _W * _A
_R, _C = 72, 128
_ONE_BITS = 0x3F800000


def _np_whctrs(anchor):
    w = anchor[2] - anchor[0] + 1.0
    h = anchor[3] - anchor[1] + 1.0
    return w, h, anchor[0] + 0.5 * (w - 1.0), anchor[1] + 0.5 * (h - 1.0)


def _np_mkanchors(ws, hs, x_ctr, y_ctr):
    ws = ws[:, None]
    hs = hs[:, None]
    return np.hstack([x_ctr - 0.5 * (ws - 1.0), y_ctr - 0.5 * (hs - 1.0),
                      x_ctr + 0.5 * (ws - 1.0), y_ctr + 0.5 * (hs - 1.0)])


def _np_anchor_planes():
    base = np.array([0.0, 0.0, _FEAT_STRIDE - 1.0, _FEAT_STRIDE - 1.0])
    w, h, xc, yc = _np_whctrs(base)
    size_ratios = (w * h) / _RATIOS
    ws = np.round(np.sqrt(size_ratios))
    hs = np.round(ws * _RATIOS)
    ratio_anchors = _np_mkanchors(ws, hs, xc, yc)
    rows = []
    for i in range(ratio_anchors.shape[0]):
        w, h, xc, yc = _np_whctrs(ratio_anchors[i])
        rows.append(_np_mkanchors(w * _SCALES, h * _SCALES, xc, yc))
    base_anchors = np.vstack(rows)
    shift = np.arange(_W) * _FEAT_STRIDE
    sx, sy = np.meshgrid(shift, shift)
    shifts = np.stack([sx.ravel(), sy.ravel(), sx.ravel(), sy.ravel()], axis=1)
    anchors = (shifts[:, None, :].astype(np.float32)
               + base_anchors[None, :, :].astype(np.float32)).reshape(-1, 4)
    aw = anchors[:, 2] - anchors[:, 0] + np.float32(1.0)
    ah = anchors[:, 3] - anchors[:, 1] + np.float32(1.0)
    acx = anchors[:, 0] + np.float32(0.5) * aw
    acy = anchors[:, 1] + np.float32(0.5) * ah
    shp = (_R, _C)
    return (aw.reshape(shp), ah.reshape(shp), acx.reshape(shp), acy.reshape(shp))


_AW, _AH, _ACX, _ACY = _np_anchor_planes()
_T72 = np.tril(np.ones((_R, _R), np.float32), -1)
_TRIU = np.triu(np.ones((_C, _C), np.float32), 1)


def _nms_body(s_ref, dx_ref, dy_ref, dw_ref, dh_ref,
              aw_ref, ah_ref, acx_ref, acy_ref, t72_ref, triu_ref,
              ox1_ref, oy1_ref, ox2_ref, oy2_ref,
              x1_ref, y1_ref, x2_ref, y2_ref, ar_ref):
    aw = aw_ref[...]
    ah = ah_ref[...]
    pcx = dx_ref[...] * aw + acx_ref[...]
    pcy = dy_ref[...] * ah + acy_ref[...]
    pw = jnp.exp(dw_ref[...]) * aw
    ph = jnp.exp(dh_ref[...]) * ah
    x1 = jnp.clip(pcx - 0.5 * pw, 0.0, _IM_W - 1.0)
    y1 = jnp.clip(pcy - 0.5 * ph, 0.0, _IM_H - 1.0)
    x2 = jnp.clip(pcx + 0.5 * pw, 0.0, _IM_W - 1.0)
    y2 = jnp.clip(pcy + 0.5 * ph, 0.0, _IM_H - 1.0)
    ws = x2 - x1 + 1.0
    hs = y2 - y1 + 1.0
    valid = (ws >= _MIN_SIZE) & (hs >= _MIN_SIZE)
    s = jnp.where(valid, s_ref[...], -jnp.inf)
    area = ws * hs

    nfin = jnp.sum((s >= 0.0).astype(jnp.float32))

    def bs_body(_, carry):
        lo, hi = carry
        mid = (lo + hi) // 2
        t = lax.bitcast_convert_type(mid, jnp.float32)
        ge = jnp.sum((s >= t).astype(jnp.float32)) >= _PRE_NMS_TOPN
        return (jnp.where(ge, mid, lo), jnp.where(ge, hi, mid))

    lo, _ = lax.fori_loop(0, 31, bs_body,
                          (jnp.int32(0), jnp.int32(_ONE_BITS)))
    v = jnp.where(nfin >= _PRE_NMS_TOPN,
                  lax.bitcast_convert_type(lo, jnp.float32),
                  -jnp.inf)

    cgt = jnp.sum((s > v).astype(jnp.float32))
    eq = (s == v).astype(jnp.float32)
    rowpre = jnp.sum(
        jnp.dot(t72_ref[...], eq, preferred_element_type=jnp.float32),
        axis=1, keepdims=True)
    lanepre = jnp.dot(eq, triu_ref[...], preferred_element_type=jnp.float32)
    rank = lanepre + rowpre
    in_top = (s > v) | ((s == v) & (rank < (_PRE_NMS_TOPN - cgt)))

    dead = jnp.float32(-3e38)
    key = jnp.where(in_top, jnp.maximum(s, jnp.float32(-1e30)), dead)

    x1_ref[...] = x1
    y1_ref[...] = y1
    x2_ref[...] = x2
    y2_ref[...] = y2
    ar_ref[...] = area

    oi = (lax.broadcasted_iota(jnp.int32, (8, 128), 0) * 128
          + lax.broadcasted_iota(jnp.int32, (8, 128), 1))
    zero8 = jnp.zeros((8, 128), jnp.float32)
    ninf = jnp.float32(-jnp.inf)
    inv_norm = jnp.float32(1.0) / jnp.float32(_IM_W - 1.0)

    def sel_body(i, carry):
        key, ox1, oy1, ox2, oy2 = carry
        ii = (lax.broadcasted_iota(jnp.int32, (_R, _C), 0) * _C
              + lax.broadcasted_iota(jnp.int32, (_R, _C), 1))
        x1v = x1_ref[...]
        y1v = y1_ref[...]
        x2v = x2_ref[...]
        y2v = y2_ref[...]
        arv = ar_ref[...]
        m = jnp.max(key)
        found = m > jnp.float32(-2e38)
        sidx = jnp.min(jnp.where(key == m, ii, jnp.int32(_N)))
        selm = ii == sidx
        bx1 = jnp.max(jnp.where(selm, x1v, ninf))
        by1 = jnp.max(jnp.where(selm, y1v, ninf))
        bx2 = jnp.max(jnp.where(selm, x2v, ninf))
        by2 = jnp.max(jnp.where(selm, y2v, ninf))
        barea = (bx2 - bx1 + 1.0) * (by2 - by1 + 1.0)
        iw = jnp.maximum(0.0,
                         jnp.minimum(bx2, x2v) - jnp.maximum(bx1, x1v) + 1.0)
        ih = jnp.maximum(0.0,
                         jnp.minimum(by2, y2v) - jnp.maximum(by1, y1v) + 1.0)
        inter = iw * ih
        iou = inter / (barea + arv - inter)
        key = jnp.where(iou > _NMS_THRESH, dead, key)
        om = (oi == i) & found
        ox1 = jnp.where(om, bx1 * inv_norm, ox1)
        oy1 = jnp.where(om, by1 * inv_norm, oy1)
        ox2 = jnp.where(om, bx2 * inv_norm, ox2)
        oy2 = jnp.where(om, by2 * inv_norm, oy2)
        return key, ox1, oy1, ox2, oy2

    _, ox1, oy1, ox2, oy2 = lax.fori_loop(
        0, _POST_NMS_TOPN, sel_body, (key, zero8, zero8, zero8, zero8))
    ox1_ref[...] = ox1
    oy1_ref[...] = oy1
    ox2_ref[...] = ox2
    oy2_ref[...] = oy2


def kernel(rpn_cls_prob, rpn_bbox_pred):
    shp = (_R, _C)
    s = rpn_cls_prob[0, :, :, _A:].reshape(shp)
    deltas = rpn_bbox_pred[0].reshape(-1, 4)
    dx = deltas[:, 0].reshape(shp)
    dy = deltas[:, 1].reshape(shp)
    dw = deltas[:, 2].reshape(shp)
    dh = deltas[:, 3].reshape(shp)
    f32 = jnp.float32
    outs = pl.pallas_call(
        _nms_body,
        out_shape=[jax.ShapeDtypeStruct((8, 128), f32)] * 4,
        scratch_shapes=[pltpu.VMEM((_R, _C), f32)] * 5,
    )(s, dx, dy, dw, dh,
      jnp.asarray(_AW), jnp.asarray(_AH), jnp.asarray(_ACX), jnp.asarray(_ACY),
      jnp.asarray(_T72), jnp.asarray(_TRIU))
    coords = [o.reshape(-1)[:_POST_NMS_TOPN] for o in outs]
    return jnp.stack(coords, axis=1)[None, :, :]


# speedup vs baseline: 6.7075x; 2.3277x over previous
import numpy as np
import jax
import jax.numpy as jnp
from jax import lax
from jax.experimental import pallas as pl
from jax.experimental.pallas import tpu as pltpu

_FEAT_STRIDE = 16
_SCALES = np.array([8.0, 16.0, 32.0])
_RATIOS = np.array([0.5, 1.0, 2.0])
_PRE_NMS_TOPN = 6000
_POST_NMS_TOPN = 300
_NMS_THRESH = 0.5
_MIN_SIZE = 16.0
_IM_H = 512.0
_IM_W = 512.0

_H = 32
_W = 32
_A = 9
_N = _H * _W * _A
_R, _C = 72, 128
_ONE_BITS = 0x3F800000


def _np_whctrs(anchor):
    w = anchor[2] - anchor[0] + 1.0
    h = anchor[3] - anchor[1] + 1.0
    return w, h, anchor[0] + 0.5 * (w - 1.0), anchor[1] + 0.5 * (h - 1.0)


def _np_mkanchors(ws, hs, x_ctr, y_ctr):
    ws = ws[:, None]
    hs = hs[:, None]
    return np.hstack([x_ctr - 0.5 * (ws - 1.0), y_ctr - 0.5 * (hs - 1.0),
                      x_ctr + 0.5 * (ws - 1.0), y_ctr + 0.5 * (hs - 1.0)])


def _np_anchor_planes():
    base = np.array([0.0, 0.0, _FEAT_STRIDE - 1.0, _FEAT_STRIDE - 1.0])
    w, h, xc, yc = _np_whctrs(base)
    size_ratios = (w * h) / _RATIOS
    ws = np.round(np.sqrt(size_ratios))
    hs = np.round(ws * _RATIOS)
    ratio_anchors = _np_mkanchors(ws, hs, xc, yc)
    rows = []
    for i in range(ratio_anchors.shape[0]):
        w, h, xc, yc = _np_whctrs(ratio_anchors[i])
        rows.append(_np_mkanchors(w * _SCALES, h * _SCALES, xc, yc))
    base_anchors = np.vstack(rows)
    shift = np.arange(_W) * _FEAT_STRIDE
    sx, sy = np.meshgrid(shift, shift)
    shifts = np.stack([sx.ravel(), sy.ravel(), sx.ravel(), sy.ravel()], axis=1)
    anchors = (shifts[:, None, :].astype(np.float32)
               + base_anchors[None, :, :].astype(np.float32)).reshape(-1, 4)
    aw = anchors[:, 2] - anchors[:, 0] + np.float32(1.0)
    ah = anchors[:, 3] - anchors[:, 1] + np.float32(1.0)
    acx = anchors[:, 0] + np.float32(0.5) * aw
    acy = anchors[:, 1] + np.float32(0.5) * ah
    shp = (_R, _C)
    return (aw.reshape(shp), ah.reshape(shp), acx.reshape(shp), acy.reshape(shp))


_AW, _AH, _ACX, _ACY = _np_anchor_planes()
_T72 = np.tril(np.ones((_R, _R), np.float32), -1)
_TRIU = np.triu(np.ones((_C, _C), np.float32), 1)


def _nms_body(s_ref, dx_ref, dy_ref, dw_ref, dh_ref,
              aw_ref, ah_ref, acx_ref, acy_ref, t72_ref, triu_ref,
              ox1_ref, oy1_ref, ox2_ref, oy2_ref,
              x1_ref, y1_ref, x2_ref, y2_ref, ar_ref, if_ref):
    aw = aw_ref[...]
    ah = ah_ref[...]
    pcx = dx_ref[...] * aw + acx_ref[...]
    pcy = dy_ref[...] * ah + acy_ref[...]
    pw = jnp.exp(dw_ref[...]) * aw
    ph = jnp.exp(dh_ref[...]) * ah
    x1 = jnp.clip(pcx - 0.5 * pw, 0.0, _IM_W - 1.0)
    y1 = jnp.clip(pcy - 0.5 * ph, 0.0, _IM_H - 1.0)
    x2 = jnp.clip(pcx + 0.5 * pw, 0.0, _IM_W - 1.0)
    y2 = jnp.clip(pcy + 0.5 * ph, 0.0, _IM_H - 1.0)
    ws = x2 - x1 + 1.0
    hs = y2 - y1 + 1.0
    valid = (ws >= _MIN_SIZE) & (hs >= _MIN_SIZE)
    s = jnp.where(valid, s_ref[...], -jnp.inf)
    area = ws * hs

    nfin = jnp.sum((s >= 0.0).astype(jnp.float32))

    def bs_body(_, carry):
        lo, hi = carry
        mid = (lo + hi) // 2
        t = lax.bitcast_convert_type(mid, jnp.float32)
        ge = jnp.sum((s >= t).astype(jnp.float32)) >= _PRE_NMS_TOPN
        return (jnp.where(ge, mid, lo), jnp.where(ge, hi, mid))

    lo, _ = lax.fori_loop(0, 31, bs_body,
                          (jnp.int32(0), jnp.int32(_ONE_BITS)))
    v = jnp.where(nfin >= _PRE_NMS_TOPN,
                  lax.bitcast_convert_type(lo, jnp.float32),
                  -jnp.inf)

    cgt = jnp.sum((s > v).astype(jnp.float32))
    eq = (s == v).astype(jnp.float32)
    rowpre = jnp.sum(
        jnp.dot(t72_ref[...], eq, preferred_element_type=jnp.float32),
        axis=1, keepdims=True)
    lanepre = jnp.dot(eq, triu_ref[...], preferred_element_type=jnp.float32)
    rank = lanepre + rowpre
    in_top = (s > v) | ((s == v) & (rank < (_PRE_NMS_TOPN - cgt)))

    dead = jnp.float32(-3e38)
    iif = (lax.broadcasted_iota(jnp.int32, (_R, _C), 0) * _C
           + lax.broadcasted_iota(jnp.int32, (_R, _C), 1)).astype(jnp.float32)
    key = jnp.where(in_top,
                    jnp.where(s >= 0.0, s, jnp.float32(-1e6) - iif),
                    dead)

    x1_ref[...] = x1
    y1_ref[...] = y1
    x2_ref[...] = x2
    y2_ref[...] = y2
    ar_ref[...] = area
    if_ref[...] = iif

    oi = (lax.broadcasted_iota(jnp.int32, (8, 128), 0) * 128
          + lax.broadcasted_iota(jnp.int32, (8, 128), 1))
    zero8 = jnp.zeros((8, 128), jnp.float32)
    ninf = jnp.float32(-jnp.inf)
    inv_norm = jnp.float32(1.0) / jnp.float32(_IM_W - 1.0)

    fn = jnp.float32(_N)

    def w_cond(carry):
        key, c, mspec, anytie, ox1, oy1, ox2, oy2 = carry
        return (c < _POST_NMS_TOPN) & (mspec > jnp.float32(-2e38))

    def w_body(carry):
        key, c, mspec, anytie, ox1, oy1, ox2, oy2 = carry
        x1v = x1_ref[...]
        y1v = y1_ref[...]
        x2v = x2_ref[...]
        y2v = y2_ref[...]
        arv = ar_ref[...]
        iiv = if_ref[...]
        eqm = key == mspec
        sidx = jnp.min(jnp.where(eqm, iiv, fn))
        sidx_hi = jnp.max(jnp.where(eqm, iiv, jnp.float32(-1.0)))
        bx1 = jnp.max(jnp.where(eqm, x1v, ninf))
        by1 = jnp.max(jnp.where(eqm, y1v, ninf))
        bx2 = jnp.max(jnp.where(eqm, x2v, ninf))
        by2 = jnp.max(jnp.where(eqm, y2v, ninf))
        m2 = jnp.max(jnp.where(eqm, dead, key))
        found = sidx < fn
        tie = sidx_hi > sidx
        anytie = anytie | tie
        barea = (bx2 - bx1 + 1.0) * (by2 - by1 + 1.0)
        iw = jnp.maximum(0.0,
                         jnp.minimum(bx2, x2v) - jnp.maximum(bx1, x1v) + 1.0)
        ih = jnp.maximum(0.0,
                         jnp.minimum(by2, y2v) - jnp.maximum(by1, y1v) + 1.0)
        inter = iw * ih
        iou = inter / (barea + arv - inter)
        key = jnp.where(iou > _NMS_THRESH, dead, key)
        om = (oi == c) & found
        ox1 = jnp.where(om, bx1 * inv_norm, ox1)
        oy1 = jnp.where(om, by1 * inv_norm, oy1)
        ox2 = jnp.where(om, bx2 * inv_norm, ox2)
        oy2 = jnp.where(om, by2 * inv_norm, oy2)
        c = c + found.astype(jnp.int32)
        mspec = jnp.where(tie, jnp.float32(jnp.inf), m2)
        return key, c, mspec, anytie, ox1, oy1, ox2, oy2

    _, _, _, anytie, sx1, sy1, sx2, sy2 = lax.while_loop(
        w_cond, w_body,
        (key, jnp.int32(0), jnp.max(key), False,
         zero8, zero8, zero8, zero8))

    def exact_loop(_):
        def body(i, carry):
            ekey, ox1, oy1, ox2, oy2 = carry
            x1v = x1_ref[...]
            y1v = y1_ref[...]
            x2v = x2_ref[...]
            y2v = y2_ref[...]
            arv = ar_ref[...]
            iiv = if_ref[...]
            m = jnp.max(ekey)
            found = m > jnp.float32(-2e38)
            eqm = ekey == m
            sidx = jnp.min(jnp.where(eqm, iiv, fn))
            selm = iiv == sidx
            bx1 = jnp.max(jnp.where(selm, x1v, ninf))
            by1 = jnp.max(jnp.where(selm, y1v, ninf))
            bx2 = jnp.max(jnp.where(selm, x2v, ninf))
            by2 = jnp.max(jnp.where(selm, y2v, ninf))
            barea = (bx2 - bx1 + 1.0) * (by2 - by1 + 1.0)
            iw = jnp.maximum(
                0.0, jnp.minimum(bx2, x2v) - jnp.maximum(bx1, x1v) + 1.0)
            ih = jnp.maximum(
                0.0, jnp.minimum(by2, y2v) - jnp.maximum(by1, y1v) + 1.0)
            inter = iw * ih
            iou = inter / (barea + arv - inter)
            ekey = jnp.where((iou > _NMS_THRESH) | selm, dead, ekey)
            om = (oi == i) & found
            ox1 = jnp.where(om, bx1 * inv_norm, ox1)
            oy1 = jnp.where(om, by1 * inv_norm, oy1)
            ox2 = jnp.where(om, bx2 * inv_norm, ox2)
            oy2 = jnp.where(om, by2 * inv_norm, oy2)
            return ekey, ox1, oy1, ox2, oy2

        _, ex1, ey1, ex2, ey2 = lax.fori_loop(
            0, _POST_NMS_TOPN, body, (key, zero8, zero8, zero8, zero8))
        return ex1, ey1, ex2, ey2

    ox1, oy1, ox2, oy2 = lax.cond(
        anytie, exact_loop, lambda _: (sx1, sy1, sx2, sy2), operand=None)
    ox1_ref[...] = ox1
    oy1_ref[...] = oy1
    ox2_ref[...] = ox2
    oy2_ref[...] = oy2


def kernel(rpn_cls_prob, rpn_bbox_pred):
    shp = (_R, _C)
    s = rpn_cls_prob[0, :, :, _A:].reshape(shp)
    deltas = rpn_bbox_pred[0].reshape(-1, 4)
    dx = deltas[:, 0].reshape(shp)
    dy = deltas[:, 1].reshape(shp)
    dw = deltas[:, 2].reshape(shp)
    dh = deltas[:, 3].reshape(shp)
    f32 = jnp.float32
    outs = pl.pallas_call(
        _nms_body,
        out_shape=[jax.ShapeDtypeStruct((8, 128), f32)] * 4,
        scratch_shapes=[pltpu.VMEM((_R, _C), f32)] * 6,
    )(s, dx, dy, dw, dh,
      jnp.asarray(_AW), jnp.asarray(_AH), jnp.asarray(_ACX), jnp.asarray(_ACY),
      jnp.asarray(_T72), jnp.asarray(_TRIU))
    coords = [o.reshape(-1)[:_POST_NMS_TOPN] for o in outs]
    return jnp.stack(coords, axis=1)[None, :, :]
